# baseline (speedup 1.0000x reference)
import functools
import jax
import jax.numpy as jnp
from jax import lax
from jax.experimental import pallas as pl
from jax.experimental.pallas import tpu as pltpu
from jax.experimental.pallas import tpu_sc as plsc

B, L, D = 4096, 200, 128
LANES = 16
NC, NS = 2, 16
NW = NC * NS
RPW = B // NW
LP = 256
NJ = 13
ND = D // LANES
CHUNK = 128
ENT = RPW * L
NCHU = ENT // CHUNK
NPH = 4
PR = RPW // NPH
PCH = NCHU // NPH
NBUF = 2
ACC_ROWS = NS * PR + 8
DUMP = NS * PR

_mesh = plsc.VectorSubcoreMesh(core_axis_name="c", subcore_axis_name="s")


@functools.partial(
    pl.kernel,
    out_type=jax.ShapeDtypeStruct((B, D), jnp.float32),
    mesh=_mesh,
    scratch_types=[
        pltpu.VMEM((NCHU, CHUNK), jnp.float32),
        pltpu.VMEM((NBUF, CHUNK, D), jnp.float32),
        pltpu.VMEM((CHUNK,), jnp.int32),
        pltpu.VMEM((CHUNK,), jnp.int32),
        pltpu.VMEM((PR, D), jnp.float32),
        pltpu.VMEM_SHARED((ACC_ROWS, D), jnp.float32),
        pltpu.SemaphoreType.DMA,
        pltpu.SemaphoreType.DMA,
        pltpu.SemaphoreType.DMA,
        pltpu.SemaphoreType.DMA,
    ],
)
def _agg(inp2_hbm, maskf_hbm, out_hbm, mflat, rowsbuf,
         rid0, rid1, outstage, accum, g0, g1, s0, s1):
    cid = lax.axis_index("c")
    sid = lax.axis_index("s")
    wid = sid * NC + cid
    base = wid * RPW
    ebase = base * L
    abase = sid * PR
    iota = lax.iota(jnp.int32, LANES)
    zvec = jnp.zeros((LANES,), jnp.float32)
    rids = (rid0, rid1)
    gsems = (g0, g1)
    ssems = (s0, s1)

    pltpu.sync_copy(maskf_hbm.at[pl.ds(wid * NCHU, NCHU)], mflat)

    def lane_gather(x, idx):
        return lax.gather(
            x, idx[:, None],
            lax.GatherDimensionNumbers(
                offset_dims=(), collapsed_slice_dims=(0,),
                start_index_map=(0,)),
            (1,), mode=lax.GatherScatterMode.PROMISE_IN_BOUNDS)

    def mkrowid(c, ph, ridbuf):
        pbase = c * CHUNK
        for k in range(CHUNK // LANES):
            p = pbase + k * LANES
            mv = mflat[c, pl.ds(k * LANES, LANES)]
            r = lax.div(p + iota, jnp.int32(L)) - ph * PR
            rid = jnp.where(mv > 0.0, abase + r, DUMP)
            ridbuf[pl.ds(k * LANES, LANES)] = rid

    def gstart(c, k):
        pltpu.async_copy(inp2_hbm.at[pl.ds(ebase + c * CHUNK, CHUNK)],
                         rowsbuf.at[k], gsems[k])

    def gwait(c, k):
        pltpu.make_async_copy(inp2_hbm.at[pl.ds(ebase + c * CHUNK, CHUNK)],
                              rowsbuf.at[k], gsems[k]).wait()

    def sstart(k):
        pltpu.async_copy(rowsbuf.at[k], accum.at[rids[k]], ssems[k],
                         add=True)

    def swait(k):
        pltpu.make_async_copy(rowsbuf.at[k], accum.at[rids[k]],
                              ssems[k]).wait()

    for ph in range(NPH):
        pbase = ph * PCH

        def ozero(i, _):
            for dv in range(ND):
                outstage[i, pl.ds(dv * LANES, LANES)] = zvec
            return 0
        lax.fori_loop(0, PR, ozero, 0)
        pltpu.sync_copy(outstage, accum.at[pl.ds(abase, PR)])

        for k in range(NBUF):
            gstart(pbase + k, k)

        def body(i, _, ph=ph, pbase=pbase):
            c0 = pbase + i * NBUF
            for k in range(NBUF):
                gwait(c0 + k, k)
                mkrowid(c0 + k, ph, rids[k])
                sstart(k)
            for k in range(NBUF):
                swait(k)

                @pl.when(c0 + NBUF + k < pbase + PCH)
                def _(k=k, c0=c0):
                    gstart(c0 + NBUF + k, k)
            return 0
        lax.fori_loop(0, PCH // NBUF, body, 0)

        for k in range(PCH % NBUF):
            c = pbase + (PCH // NBUF) * NBUF + k
            gwait(c, k)
            mkrowid(c, ph, rids[k])
            sstart(k)
        for k in range(PCH % NBUF):
            swait(k)

        pltpu.sync_copy(accum.at[pl.ds(abase, PR)], outstage)

        def dbody(io, _, ph=ph):
            for k in range(16):
                i = io * 16 + k
                mrow = ph * PCH + io * 25 + (k * L) // CHUNK
                col = (k * L) % CHUNK
                s = zvec
                a0 = (col // LANES) * LANES
                a = a0
                while a < col + L:
                    v = mflat[mrow + a // CHUNK, pl.ds(a % CHUNK, LANES)]
                    if a < col:
                        v = jnp.where(iota >= col - a, v, 0.0)
                    if a + LANES > col + L:
                        v = jnp.where(iota < col + L - a, v, 0.0)
                    s = s + v
                    a += LANES
                for kk in (8, 4, 2, 1):
                    s = s + lane_gather(s, iota ^ kk)
                for dv in range(ND):
                    outstage[i, pl.ds(dv * LANES, LANES)] = (
                        outstage[i, pl.ds(dv * LANES, LANES)] / s)
            return 0
        lax.fori_loop(0, PR // 16, dbody, 0)
        pltpu.sync_copy(outstage, out_hbm.at[pl.ds(base + ph * PR, PR)])


def kernel(inputs, mask):
    mflat = mask.astype(jnp.float32).reshape(B * L // CHUNK, CHUNK)
    inp2 = inputs.reshape(B * L, D)
    return _agg(inp2, mflat)


# speedup vs baseline: 1.1518x; 1.1518x over previous
import functools
import jax
import jax.numpy as jnp
from jax import lax
from jax.experimental import pallas as pl
from jax.experimental.pallas import tpu as pltpu
from jax.experimental.pallas import tpu_sc as plsc

B, L, D = 4096, 200, 128
LANES = 16
NC, NS = 2, 16
NW = NC * NS
RPW = B // NW
LP = 256
NJ = LP // LANES
ND = D // LANES
CHUNK = 128
ENT = RPW * L
NCHU = ENT // CHUNK
NPH = 4
PR = RPW // NPH
PCH = NCHU // NPH
ACC_ROWS = NS * PR + 8
DUMP = NS * PR

_mesh = plsc.VectorSubcoreMesh(core_axis_name="c", subcore_axis_name="s")


@functools.partial(
    pl.kernel,
    out_type=jax.ShapeDtypeStruct((B, D), jnp.float32),
    mesh=_mesh,
    scratch_types=[
        pltpu.VMEM((RPW, LP), jnp.float32),
        pltpu.VMEM((NCHU, CHUNK), jnp.float32),
        pltpu.VMEM((2, CHUNK, D), jnp.float32),
        pltpu.VMEM((CHUNK,), jnp.int32),
        pltpu.VMEM((CHUNK,), jnp.int32),
        pltpu.VMEM((RPW, LANES), jnp.float32),
        pltpu.VMEM((RPW, D), jnp.float32),
        pltpu.VMEM_SHARED((ACC_ROWS, D), jnp.float32),
        pltpu.SemaphoreType.DMA,
        pltpu.SemaphoreType.DMA,
        pltpu.SemaphoreType.DMA,
        pltpu.SemaphoreType.DMA,
        pltpu.SemaphoreType.DMA,
    ],
)
def _agg(inp2_hbm, maskp_hbm, maskf_hbm, out_hbm, maskbuf, mflat, rowsbuf,
         rid0, rid1, countbuf, outbuf, accum, gsem0, gsem1, ssem0, ssem1,
         msem):
    cid = lax.axis_index("c")
    sid = lax.axis_index("s")
    wid = sid * NC + cid
    base = wid * RPW
    ebase = base * L
    abase = sid * PR
    iota = lax.iota(jnp.int32, LANES)
    zvec = jnp.zeros((LANES,), jnp.float32)

    pltpu.async_copy(maskp_hbm.at[pl.ds(base, RPW)], maskbuf, msem)
    pltpu.async_copy(maskf_hbm.at[pl.ds(wid * NCHU, NCHU)], mflat, gsem0)

    def zbody(r, _):
        for dv in range(ND):
            outbuf[r, pl.ds(dv * LANES, LANES)] = zvec
        return 0
    lax.fori_loop(0, RPW, zbody, 0)
    pltpu.make_async_copy(maskp_hbm.at[pl.ds(base, RPW)], maskbuf, msem).wait()
    pltpu.make_async_copy(maskf_hbm.at[pl.ds(wid * NCHU, NCHU)], mflat,
                          gsem0).wait()

    def lane_gather(x, idx):
        return lax.gather(
            x, idx[:, None],
            lax.GatherDimensionNumbers(
                offset_dims=(), collapsed_slice_dims=(0,),
                start_index_map=(0,)),
            (1,), mode=lax.GatherScatterMode.PROMISE_IN_BOUNDS)

    def crow(r, _):
        s = zvec
        for j in range(NJ):
            s = s + maskbuf[r, pl.ds(j * LANES, LANES)]
        for k in (8, 4, 2, 1):
            s = s + lane_gather(s, iota ^ k)
        countbuf[r, :] = s
        return 0
    lax.fori_loop(0, RPW, crow, 0)

    def mkrowid(c, ph, ridbuf):
        pbase = c * CHUNK
        for k in range(CHUNK // LANES):
            p = pbase + k * LANES
            mv = mflat[c, pl.ds(k * LANES, LANES)]
            r = lax.div(p + iota, jnp.int32(L)) - ph * PR
            rid = jnp.where(mv > 0.0, abase + r, DUMP)
            ridbuf[pl.ds(k * LANES, LANES)] = rid

    def gstart(c, slot, gsem):
        pltpu.async_copy(inp2_hbm.at[pl.ds(ebase + c * CHUNK, CHUNK)],
                         rowsbuf.at[slot], gsem)

    def gwait(c, slot, gsem):
        pltpu.make_async_copy(inp2_hbm.at[pl.ds(ebase + c * CHUNK, CHUNK)],
                              rowsbuf.at[slot], gsem).wait()

    def sstart(slot, ridbuf, ssem):
        pltpu.async_copy(rowsbuf.at[slot], accum.at[ridbuf], ssem, add=True)

    def swait(slot, ridbuf, ssem):
        pltpu.make_async_copy(rowsbuf.at[slot], accum.at[ridbuf], ssem).wait()

    for ph in range(NPH):
        pltpu.sync_copy(outbuf.at[pl.ds(ph * PR, PR)],
                        accum.at[pl.ds(abase, PR)])

        gstart(ph * PCH, 0, gsem0)

        def body(i, _, ph=ph):
            c0 = ph * PCH + i * 2
            gwait(c0, 0, gsem0)
            mkrowid(c0, ph, rid0)
            sstart(0, rid0, ssem0)

            @pl.when(i > 0)
            def _():
                swait(1, rid1, ssem1)
            gstart(c0 + 1, 1, gsem1)

            gwait(c0 + 1, 1, gsem1)
            mkrowid(c0 + 1, ph, rid1)
            sstart(1, rid1, ssem1)
            swait(0, rid0, ssem0)

            @pl.when(i + 1 < PCH // 2)
            def _():
                gstart(c0 + 2, 0, gsem0)
            return 0
        lax.fori_loop(0, PCH // 2, body, 0)
        swait(1, rid1, ssem1)

        pltpu.sync_copy(accum.at[pl.ds(abase, PR)],
                        outbuf.at[pl.ds(ph * PR, PR)])

    def dbody(r, _):
        c16 = countbuf[r, :]
        for dv in range(ND):
            outbuf[r, pl.ds(dv * LANES, LANES)] = (
                outbuf[r, pl.ds(dv * LANES, LANES)] / c16)
        return 0
    lax.fori_loop(0, RPW, dbody, 0)
    pltpu.sync_copy(outbuf, out_hbm.at[pl.ds(base, RPW)])


def kernel(inputs, mask):
    maskf = mask.astype(jnp.float32)
    maskp = jnp.pad(maskf, ((0, 0), (0, LP - L)))
    mflat = maskf.reshape(B * L // CHUNK, CHUNK)
    inp2 = inputs.reshape(B * L, D)
    return _agg(inp2, maskp, mflat)
